# baseline (speedup 1.0000x reference)
import functools

import jax
import jax.numpy as jnp
from jax import lax
from jax.experimental import pallas as pl
from jax.experimental.pallas import tpu as pltpu
from jax.experimental.pallas import tpu_sc as plsc

N = 10000
E = 320000
F = 128
NC, NS = 2, 16
NW = NC * NS
EB = 64
NG = 160
IC = 16
EROWS = NW * NG
EPAD = EROWS * EB
NAE = 10240
NBUF = 4
ZCH = 8
NZK = NAE // (ZCH * NS)

_mesh = plsc.VectorSubcoreMesh(core_axis_name="c", subcore_axis_name="s")


@functools.partial(
    pl.kernel,
    out_type=jax.ShapeDtypeStruct((NC, NAE, F), jnp.float32),
    mesh=_mesh,
    scratch_types=[
        pltpu.VMEM((2, IC, EB), jnp.int32),
        pltpu.VMEM((2, IC, EB), jnp.int32),
        [pltpu.VMEM((EB, F), jnp.float32)] * NBUF,
        pltpu.VMEM((ZCH, F), jnp.float32),
        pltpu.VMEM_SHARED((NAE, F), jnp.float32),
        [pltpu.SemaphoreType.DMA] * NBUF,
        pltpu.SemaphoreType.DMA,
    ],
)
def _sc_agg(x_hbm, src_hbm, dst_hbm, out_hbm,
            sidx, didx, rows, zbuf, acc, sems, sem_i):
    cid = lax.axis_index("c")
    sid = lax.axis_index("s")
    wid = cid * NS + sid
    rbase = wid * NG

    pltpu.sync_copy(src_hbm.at[pl.ds(rbase, IC)], sidx.at[0])
    pltpu.sync_copy(dst_hbm.at[pl.ds(rbase, IC)], didx.at[0])
    pltpu.async_copy(src_hbm.at[pl.ds(rbase + IC, IC)], sidx.at[1], sem_i)
    pltpu.async_copy(dst_hbm.at[pl.ds(rbase + IC, IC)], didx.at[1], sem_i)

    zvec = jnp.zeros((16,), jnp.float32)
    for r in range(ZCH):
        for c in range(F // 16):
            zbuf[r, pl.ds(c * 16, 16)] = zvec

    def _zchunk(k0, carry):
        for b in range(NBUF):
            k = k0 * NBUF + b

            @pl.when(k >= NBUF)
            def _():
                pltpu.make_async_copy(zbuf, acc.at[pl.ds(0, ZCH)], sems[b]).wait()

            row = (k * NS + sid) * ZCH
            pltpu.async_copy(zbuf, acc.at[pl.ds(row, ZCH)], sems[b])
        return carry

    lax.fori_loop(0, NZK // NBUF, _zchunk, 0)
    for b in range(NBUF):
        pltpu.make_async_copy(zbuf, acc.at[pl.ds(0, ZCH)], sems[b]).wait()
    plsc.subcore_barrier()

    def _gather(gq, b):
        slot = (gq // IC) % 2
        r = gq % IC
        pltpu.async_copy(x_hbm.at[sidx.at[slot, r]], rows[b], sems[b])

    for b in range(2):
        _gather(b, b)

    def _pipe(i, carry):
        h = (i * NBUF) // IC

        @pl.when((i % (IC // NBUF) == IC // NBUF - 1) & (h < NG // IC - 1))
        def _():
            pltpu.make_async_copy(src_hbm.at[pl.ds(0, IC)], sidx.at[0], sem_i).wait()
            pltpu.make_async_copy(dst_hbm.at[pl.ds(0, IC)], didx.at[0], sem_i).wait()

        for b in range(NBUF):
            g = i * NBUF + b
            slot = (g // IC) % 2
            r = g % IC
            pltpu.make_async_copy(x_hbm.at[sidx.at[0, 0]], rows[b], sems[b]).wait()
            pltpu.async_copy(rows[b], acc.at[didx.at[slot, r]], sems[b], add=True)

            b2 = (b + 2) % NBUF

            @pl.when(g >= 2)
            def _():
                pltpu.make_async_copy(rows[b2], acc.at[pl.ds(0, EB)], sems[b2]).wait()

            @pl.when(g + 2 < NG)
            def _():
                _gather(g + 2, b2)

        @pl.when((i % (IC // NBUF) == 0) & (h >= 1) & (h < NG // IC - 1))
        def _():
            rb = rbase + (h + 1) * IC
            pltpu.async_copy(src_hbm.at[pl.ds(rb, IC)], sidx.at[(h + 1) % 2], sem_i)
            pltpu.async_copy(dst_hbm.at[pl.ds(rb, IC)], didx.at[(h + 1) % 2], sem_i)

        return carry

    lax.fori_loop(0, NG // NBUF, _pipe, 0)

    for b in ((NG - 2) % NBUF, (NG - 1) % NBUF):
        pltpu.make_async_copy(rows[b], acc.at[pl.ds(0, EB)], sems[b]).wait()
    plsc.subcore_barrier()

    def _ochunk(k0, carry):
        for b in range(NBUF):
            k = k0 * NBUF + b

            @pl.when(k >= NBUF)
            def _():
                pltpu.make_async_copy(acc.at[pl.ds(0, ZCH)],
                                      out_hbm.at[cid, pl.ds(0, ZCH)], sems[b]).wait()

            row = (k * NS + sid) * ZCH
            pltpu.async_copy(acc.at[pl.ds(row, ZCH)],
                             out_hbm.at[cid, pl.ds(row, ZCH)], sems[b])
        return carry

    lax.fori_loop(0, NZK // NBUF, _ochunk, 0)
    for b in range(NBUF):
        pltpu.make_async_copy(acc.at[pl.ds(0, ZCH)],
                              out_hbm.at[cid, pl.ds(0, ZCH)], sems[b]).wait()


def _tc_layer_call(y, x, W, Ws, b, *, final):
    M, Fin = x.shape
    Fo = W.shape[1]
    BM = 1000

    def body(ya_ref, yb_ref, x_ref, W_ref, Ws_ref, b_ref, o_ref):
        ys = ya_ref[0] + yb_ref[0]
        acc = jnp.dot(ys, W_ref[...], preferred_element_type=jnp.float32)
        acc += jnp.dot(x_ref[...], Ws_ref[...], preferred_element_type=jnp.float32)
        logits = acc + b_ref[...]
        if final:
            m = jnp.max(logits, axis=-1, keepdims=True)
            z = logits - m
            lse = jnp.log(jnp.sum(jnp.exp(z), axis=-1, keepdims=True))
            o_ref[...] = z - lse
        else:
            o_ref[...] = logits

    return pl.pallas_call(
        body,
        grid=(M // BM,),
        in_specs=[
            pl.BlockSpec((1, BM, Fin), lambda i: (0, i, 0)),
            pl.BlockSpec((1, BM, Fin), lambda i: (1, i, 0)),
            pl.BlockSpec((BM, Fin), lambda i: (i, 0)),
            pl.BlockSpec((Fin, Fo), lambda i: (0, 0)),
            pl.BlockSpec((Fin, Fo), lambda i: (0, 0)),
            pl.BlockSpec((1, Fo), lambda i: (0, 0)),
        ],
        out_specs=pl.BlockSpec((BM, Fo), lambda i: (i, 0)),
        out_shape=jax.ShapeDtypeStruct((M, Fo), jnp.float32),
    )(y, y, x, W, Ws, b.reshape(1, Fo))


def kernel(fea, edge_index, W_in, Ws_in, b_in, W_mid, Ws_mid, b_mid,
           W_out, Ws_out, b_out):
    pad = EPAD - E
    src = jnp.concatenate([edge_index[0], jnp.zeros((pad,), jnp.int32)])
    dst = jnp.concatenate([edge_index[1], jnp.full((pad,), N, jnp.int32)])
    src = src.reshape(EROWS, EB)
    dst = dst.reshape(EROWS, EB)
    y = _sc_agg(fea, src, dst)
    x1 = _tc_layer_call(y, fea, W_in, Ws_in, b_in, final=False)
    y = _sc_agg(x1, src, dst)
    x2 = _tc_layer_call(y, x1, W_mid, Ws_mid, b_mid, final=False)
    y = _sc_agg(x2, src, dst)
    return _tc_layer_call(y, x2, W_out, Ws_out, b_out, final=True)
